# Initial kernel scaffold; baseline (speedup 1.0000x reference)
import functools

import jax
import jax.numpy as jnp
from jax import lax
from jax.experimental import pallas as pl
from jax.experimental.pallas import tpu as pltpu
from jax.experimental.pallas import tpu_sc as plsc

N = 10000
NPAD = 10240
T = 12
C = 128
NNZ = 160000
NCORES = 2
NSUB = 16
EBLK = 128
NBLK = 79
EPT = EBLK * NBLK
NNZ_PAD = EPT * NSUB
CHUNKS_PER_CORE = T // NCORES
ROWS_PER_TILE = NPAD // NSUB


def _sc_spmm_body(xrows, rows, cols, vals, z, rows_v, cols_v, vals_v,
                  idxbuf, rowbuf, gbuf, zerobuf, acc, sem):
    c = lax.axis_index("c")
    s = lax.axis_index("s")
    e0 = s * EPT
    pltpu.sync_copy(rows.at[pl.ds(e0, EPT)], rows_v)
    pltpu.sync_copy(cols.at[pl.ds(e0, EPT)], cols_v)
    pltpu.sync_copy(vals.at[pl.ds(e0, EPT)], vals_v)

    def _mul_t(i, _):
        cols_v[pl.ds(i * 16, 16)] = cols_v[pl.ds(i * 16, 16)] * T
        return 0
    lax.fori_loop(0, EPT // 16, _mul_t, 0)

    def _zfill(i, _):
        for k in range(C // 16):
            zerobuf[i, pl.ds(k * 16, 16)] = jnp.zeros((16,), jnp.float32)
        return 0
    lax.fori_loop(0, EBLK, _zfill, 0)

    for jj in range(CHUNKS_PER_CORE):
        j = c * CHUNKS_PER_CORE + jj

        for p in range(ROWS_PER_TILE // EBLK):
            pltpu.sync_copy(zerobuf, acc.at[pl.ds(s * ROWS_PER_TILE + p * EBLK, EBLK)])
        plsc.subcore_barrier()

        def _block(b, _):
            base = b * EBLK

            def _mkidx(g, _):
                idxbuf[pl.ds(g * 16, 16)] = cols_v[pl.ds(base + g * 16, 16)] + j
                rowbuf[pl.ds(g * 16, 16)] = rows_v[pl.ds(base + g * 16, 16)]
                return 0
            lax.fori_loop(0, EBLK // 16, _mkidx, 0)

            pltpu.async_copy(xrows.at[idxbuf], gbuf, sem).wait()

            def _scale(g, _):
                v16 = vals_v[pl.ds(base + g * 16, 16)]
                for i in range(16):
                    vb = jnp.broadcast_to(v16[i:i + 1], (16,))
                    e = g * 16 + i
                    for k in range(C // 16):
                        gbuf[e, pl.ds(k * 16, 16)] = gbuf[e, pl.ds(k * 16, 16)] * vb
                return 0
            lax.fori_loop(0, EBLK // 16, _scale, 0)

            pltpu.sync_copy(gbuf, acc.at[rowbuf], add=True)
            return 0
        lax.fori_loop(0, NBLK, _block, 0)

        plsc.subcore_barrier()
        pltpu.sync_copy(acc.at[pl.ds(s * ROWS_PER_TILE, ROWS_PER_TILE)],
                        z.at[j, pl.ds(s * ROWS_PER_TILE, ROWS_PER_TILE)])
        plsc.subcore_barrier()


def _sc_spmm(xrows, rows, cols, vals):
    mesh = plsc.VectorSubcoreMesh(core_axis_name="c", subcore_axis_name="s",
                                  num_cores=NCORES, num_subcores=NSUB)
    f = pl.kernel(
        _sc_spmm_body,
        out_type=jax.ShapeDtypeStruct((T, NPAD, C), jnp.float32),
        mesh=mesh,
        scratch_types=[
            pltpu.VMEM((EPT,), jnp.int32),
            pltpu.VMEM((EPT,), jnp.int32),
            pltpu.VMEM((EPT,), jnp.float32),
            pltpu.VMEM((EBLK,), jnp.int32),
            pltpu.VMEM((EBLK,), jnp.int32),
            pltpu.VMEM((EBLK, C), jnp.float32),
            pltpu.VMEM((EBLK, C), jnp.float32),
            pltpu.VMEM_SHARED((NPAD, C), jnp.float32),
            pltpu.SemaphoreType.DMA,
        ],
    )
    return f(xrows, rows, cols, vals)


def _tc_body(z_ref, w_ref, b_ref, x_ref, o_ref):
    g = lax.dot_general(w_ref[...], z_ref[...], (((0,), (1,)), ((), ())),
                        preferred_element_type=jnp.float32)
    o_ref[...] = g + b_ref[...] + x_ref[...]


def _tc_post(z_r, weight, bias, x2p):
    rb = T * C
    grid = NPAD // C
    return pl.pallas_call(
        _tc_body,
        grid=(grid,),
        in_specs=[
            pl.BlockSpec((rb, C), lambda i: (i, 0)),
            pl.BlockSpec((C, C), lambda i: (0, 0)),
            pl.BlockSpec((C, 1), lambda i: (0, 0)),
            pl.BlockSpec((C, rb), lambda i: (0, i)),
        ],
        out_specs=pl.BlockSpec((C, rb), lambda i: (0, i)),
        out_shape=jax.ShapeDtypeStruct((C, NPAD * T), jnp.float32),
    )(z_r, weight, bias, x2p)


def kernel(x, weight, bias, filter_rows, filter_cols, filter_vals):
    xrows = x.reshape(N * T, C)
    rows = jnp.concatenate(
        [filter_rows.astype(jnp.int32),
         jnp.full((NNZ_PAD - NNZ,), NPAD - 1, jnp.int32)])
    cols = jnp.concatenate(
        [filter_cols.astype(jnp.int32), jnp.zeros((NNZ_PAD - NNZ,), jnp.int32)])
    vals = jnp.concatenate(
        [filter_vals, jnp.zeros((NNZ_PAD - NNZ,), jnp.float32)])

    z_t = _sc_spmm(xrows, rows, cols, vals)
    z_r = z_t.transpose(1, 0, 2).reshape(NPAD * T, C)

    x2p = jnp.pad(x.reshape(C, N * T), ((0, 0), (0, (NPAD - N) * T)))
    out2 = _tc_post(z_r, weight, bias.reshape(C, 1), x2p)
    return out2[:, :N * T].reshape(1, C, T, N)


# speedup vs baseline: 1.9851x; 1.9851x over previous
import functools

import jax
import jax.numpy as jnp
from jax import lax
from jax.experimental import pallas as pl
from jax.experimental.pallas import tpu as pltpu
from jax.experimental.pallas import tpu_sc as plsc

N = 10000
NPAD = 10240
T = 12
C = 128
NNZ = 160000
NCORES = 2
NSUB = 16
EBLK = 128
NBLK = 79
EPT = EBLK * NBLK
NNZ_PAD = EPT * NSUB
CHUNKS_PER_CORE = T // NCORES
ROWS_PER_TILE = NPAD // NSUB


def _sc_spmm_body(xrows, rows, cols, vals, z, rows_v, cols_v, vals_v,
                  idxbuf, rowbuf, gbuf, acc, sem):
    c = lax.axis_index("c")
    s = lax.axis_index("s")
    e0 = s * EPT
    pltpu.sync_copy(rows.at[pl.ds(e0, EPT)], rows_v)
    pltpu.sync_copy(cols.at[pl.ds(e0, EPT)], cols_v)
    pltpu.sync_copy(vals.at[pl.ds(e0, EPT)], vals_v)

    def _mul_t(i, _):
        cols_v[pl.ds(i * 16, 16)] = cols_v[pl.ds(i * 16, 16)] * T
        return 0
    lax.fori_loop(0, EPT // 16, _mul_t, 0)

    for jj in range(CHUNKS_PER_CORE):
        j = c * CHUNKS_PER_CORE + jj

        def _zfill(i, _):
            for k in range(C // 16):
                gbuf[i, pl.ds(k * 16, 16)] = jnp.zeros((16,), jnp.float32)
            return 0
        lax.fori_loop(0, EBLK, _zfill, 0)
        for p in range(ROWS_PER_TILE // EBLK):
            pltpu.sync_copy(gbuf, acc.at[pl.ds(s * ROWS_PER_TILE + p * EBLK, EBLK)])
        plsc.subcore_barrier()

        def _block(b, _):
            base = b * EBLK

            def _mkidx(g, _):
                idxbuf[pl.ds(g * 16, 16)] = cols_v[pl.ds(base + g * 16, 16)] + j
                rowbuf[pl.ds(g * 16, 16)] = rows_v[pl.ds(base + g * 16, 16)]
                return 0
            lax.fori_loop(0, EBLK // 16, _mkidx, 0)

            pltpu.async_copy(xrows.at[idxbuf], gbuf, sem).wait()

            def _scale(g, _):
                v16 = vals_v[pl.ds(base + g * 16, 16)]
                for i in range(16):
                    vb = jnp.broadcast_to(v16[i:i + 1], (16,))
                    e = g * 16 + i
                    for k in range(C // 16):
                        gbuf[e, pl.ds(k * 16, 16)] = gbuf[e, pl.ds(k * 16, 16)] * vb
                return 0
            lax.fori_loop(0, EBLK // 16, _scale, 0)

            pltpu.sync_copy(gbuf, acc.at[rowbuf], add=True)
            return 0
        lax.fori_loop(0, NBLK, _block, 0)

        plsc.subcore_barrier()
        pltpu.sync_copy(acc.at[pl.ds(s * ROWS_PER_TILE, ROWS_PER_TILE)],
                        z.at[j, pl.ds(s * ROWS_PER_TILE, ROWS_PER_TILE)])
        plsc.subcore_barrier()


def _sc_spmm(xrows, rows, cols, vals):
    mesh = plsc.VectorSubcoreMesh(core_axis_name="c", subcore_axis_name="s",
                                  num_cores=NCORES, num_subcores=NSUB)
    f = pl.kernel(
        _sc_spmm_body,
        out_type=jax.ShapeDtypeStruct((T, NPAD, C), jnp.float32),
        mesh=mesh,
        scratch_types=[
            pltpu.VMEM((EPT,), jnp.int32),
            pltpu.VMEM((EPT,), jnp.int32),
            pltpu.VMEM((EPT,), jnp.float32),
            pltpu.VMEM((EBLK,), jnp.int32),
            pltpu.VMEM((EBLK,), jnp.int32),
            pltpu.VMEM((EBLK, C), jnp.float32),
            pltpu.VMEM_SHARED((NPAD, C), jnp.float32),
            pltpu.SemaphoreType.DMA,
        ],
    )
    return f(xrows, rows, cols, vals)


def _tc_body(z_ref, w_ref, b_ref, x_ref, o_ref):
    g = lax.dot_general(w_ref[...], z_ref[...], (((0,), (1,)), ((), ())),
                        preferred_element_type=jnp.float32)
    o_ref[...] = g + b_ref[...] + x_ref[...]


def _tc_post(z_r, weight, bias, x2p):
    rb = T * C
    grid = NPAD // C
    return pl.pallas_call(
        _tc_body,
        grid=(grid,),
        in_specs=[
            pl.BlockSpec((rb, C), lambda i: (i, 0)),
            pl.BlockSpec((C, C), lambda i: (0, 0)),
            pl.BlockSpec((C, 1), lambda i: (0, 0)),
            pl.BlockSpec((C, rb), lambda i: (0, i)),
        ],
        out_specs=pl.BlockSpec((C, rb), lambda i: (0, i)),
        out_shape=jax.ShapeDtypeStruct((C, NPAD * T), jnp.float32),
    )(z_r, weight, bias, x2p)


def kernel(x, weight, bias, filter_rows, filter_cols, filter_vals):
    xrows = x.reshape(N * T, C)
    rows = jnp.concatenate(
        [filter_rows.astype(jnp.int32),
         jnp.full((NNZ_PAD - NNZ,), NPAD - 1, jnp.int32)])
    cols = jnp.concatenate(
        [filter_cols.astype(jnp.int32), jnp.zeros((NNZ_PAD - NNZ,), jnp.int32)])
    vals = jnp.concatenate(
        [filter_vals, jnp.zeros((NNZ_PAD - NNZ,), jnp.float32)])

    z_t = _sc_spmm(xrows, rows, cols, vals)
    z_r = z_t.transpose(1, 0, 2).reshape(NPAD * T, C)

    x2p = jnp.pad(x.reshape(C, N * T), ((0, 0), (0, (NPAD - N) * T)))
    out2 = _tc_post(z_r, weight, bias.reshape(C, 1), x2p)
    return out2[:, :N * T].reshape(1, C, T, N)


# speedup vs baseline: 2.0133x; 1.0142x over previous
import functools

import jax
import jax.numpy as jnp
from jax import lax
from jax.experimental import pallas as pl
from jax.experimental.pallas import tpu as pltpu
from jax.experimental.pallas import tpu_sc as plsc

N = 10000
NPAD = 10240
T = 12
C = 128
NNZ = 160000
NCORES = 2
NSUB = 16
EBLK = 128
NBLK = 80
EPT = EBLK * NBLK
NNZ_PAD = EPT * NSUB
CHUNKS_PER_CORE = T // NCORES
ROWS_PER_TILE = NPAD // NSUB


def _sc_spmm_body(xrows, rows, cols, vals, z,
                  gbuf0, gbuf1, colsb0, colsb1, valsb0, valsb1,
                  rowsb0, rowsb1, rowscat0, rowscat1, acc,
                  ge0, ge1, se0, se1, lc0, lc1, lv0, lv1, lr0, lr1):
    c = lax.axis_index("c")
    s = lax.axis_index("s")
    e0 = s * EPT
    gbufs = [gbuf0, gbuf1]
    colsbx = [colsb0, colsb1]
    valsbx = [valsb0, valsb1]
    rowsbx = [rowsb0, rowsb1]
    rowscats = [rowscat0, rowscat1]
    ges = [ge0, ge1]
    ses = [se0, se1]
    lcs = [lc0, lc1]
    lvs = [lv0, lv1]
    lrs = [lr0, lr1]

    def fire_load(bv, p):
        off = e0 + bv * EBLK
        pltpu.async_copy(cols.at[pl.ds(off, EBLK)], colsbx[p], lcs[p])
        pltpu.async_copy(vals.at[pl.ds(off, EBLK)], valsbx[p], lvs[p])
        pltpu.async_copy(rows.at[pl.ds(off, EBLK)], rowsbx[p], lrs[p])

    def wait_load(p):
        pltpu.make_async_copy(cols.at[pl.ds(0, EBLK)], colsbx[p], lcs[p]).wait()
        pltpu.make_async_copy(vals.at[pl.ds(0, EBLK)], valsbx[p], lvs[p]).wait()
        pltpu.make_async_copy(rows.at[pl.ds(0, EBLK)], rowsbx[p], lrs[p]).wait()

    def fire_gather(p):
        pltpu.async_copy(xrows.at[colsbx[p]], gbufs[p], ges[p])

    def wait_gather(p):
        pltpu.make_async_copy(xrows.at[colsbx[p]], gbufs[p], ges[p]).wait()

    def fire_scatter(p):
        pltpu.async_copy(gbufs[p], acc.at[rowscats[p]], ses[p], add=True)

    def wait_scatter(p):
        pltpu.make_async_copy(gbufs[p], acc.at[rowscats[p]], ses[p]).wait()

    def chunk_body(jj, _):
        j = c * CHUNKS_PER_CORE + jj

        def idx_compute(p):
            def f(g, _):
                colsbx[p][pl.ds(g * 16, 16)] = colsbx[p][pl.ds(g * 16, 16)] * T + j
                return 0
            lax.fori_loop(0, EBLK // 16, f, 0)

        def scale(p):
            def g_body(g, _):
                rowscats[p][pl.ds(g * 16, 16)] = rowsbx[p][pl.ds(g * 16, 16)]
                v16 = valsbx[p][pl.ds(g * 16, 16)]
                for i in range(16):
                    vb = jnp.broadcast_to(v16[i:i + 1], (16,))
                    e = g * 16 + i
                    for k in range(C // 16):
                        gbufs[p][e, pl.ds(k * 16, 16)] = (
                            gbufs[p][e, pl.ds(k * 16, 16)] * vb)
                return 0
            lax.fori_loop(0, EBLK // 16, g_body, 0)

        def slot(bv, p, wait_prev_scatter, next_gather, next_load):
            q = 1 - p
            if next_gather:
                wait_load(q)
                idx_compute(q)
                if wait_prev_scatter:
                    wait_scatter(q)
                fire_gather(q)
            wait_gather(p)
            scale(p)
            fire_scatter(p)
            if next_load:
                fire_load(bv + 2, p)

        def _zfill(i, _):
            for k in range(C // 16):
                gbuf0[i, pl.ds(k * 16, 16)] = jnp.zeros((16,), jnp.float32)
            return 0
        lax.fori_loop(0, EBLK, _zfill, 0)
        for p in range(ROWS_PER_TILE // EBLK):
            pltpu.sync_copy(gbuf0, acc.at[pl.ds(s * ROWS_PER_TILE + p * EBLK, EBLK)])
        plsc.subcore_barrier()

        fire_load(0, 0)
        fire_load(1, 1)
        wait_load(0)
        idx_compute(0)
        fire_gather(0)
        slot(0, 0, wait_prev_scatter=False, next_gather=True, next_load=True)
        slot(1, 1, wait_prev_scatter=True, next_gather=True, next_load=True)

        def pair(ii, _):
            bv = 2 * ii + 2
            slot(bv, 0, wait_prev_scatter=True, next_gather=True, next_load=True)
            slot(bv + 1, 1, wait_prev_scatter=True, next_gather=True, next_load=True)
            return 0
        lax.fori_loop(0, (NBLK - 4) // 2, pair, 0)

        slot(NBLK - 2, 0, wait_prev_scatter=True, next_gather=True, next_load=False)
        slot(NBLK - 1, 1, wait_prev_scatter=False, next_gather=False, next_load=False)
        wait_scatter(0)
        wait_scatter(1)

        plsc.subcore_barrier()
        pltpu.sync_copy(acc.at[pl.ds(s * ROWS_PER_TILE, ROWS_PER_TILE)],
                        z.at[jj + c * CHUNKS_PER_CORE, pl.ds(s * ROWS_PER_TILE, ROWS_PER_TILE)])
        plsc.subcore_barrier()
        return 0

    lax.fori_loop(0, CHUNKS_PER_CORE, chunk_body, 0)


def _sc_spmm(xrows, rows, cols, vals):
    mesh = plsc.VectorSubcoreMesh(core_axis_name="c", subcore_axis_name="s",
                                  num_cores=NCORES, num_subcores=NSUB)
    f = pl.kernel(
        _sc_spmm_body,
        out_type=jax.ShapeDtypeStruct((T, NPAD, C), jnp.float32),
        mesh=mesh,
        scratch_types=[
            pltpu.VMEM((EBLK, C), jnp.float32),
            pltpu.VMEM((EBLK, C), jnp.float32),
            pltpu.VMEM((EBLK,), jnp.int32),
            pltpu.VMEM((EBLK,), jnp.int32),
            pltpu.VMEM((EBLK,), jnp.float32),
            pltpu.VMEM((EBLK,), jnp.float32),
            pltpu.VMEM((EBLK,), jnp.int32),
            pltpu.VMEM((EBLK,), jnp.int32),
            pltpu.VMEM((EBLK,), jnp.int32),
            pltpu.VMEM((EBLK,), jnp.int32),
            pltpu.VMEM_SHARED((NPAD, C), jnp.float32),
        ] + [pltpu.SemaphoreType.DMA] * 10,
    )
    return f(xrows, rows, cols, vals)


def _tc_body(z_ref, w_ref, b_ref, x_ref, o_ref):
    g = lax.dot_general(w_ref[...], z_ref[...], (((0,), (1,)), ((), ())),
                        preferred_element_type=jnp.float32)
    o_ref[...] = g + b_ref[...] + x_ref[...]


def _tc_post(z_r, weight, bias, x2p):
    rb = T * C
    grid = NPAD // C
    return pl.pallas_call(
        _tc_body,
        grid=(grid,),
        in_specs=[
            pl.BlockSpec((rb, C), lambda i: (i, 0)),
            pl.BlockSpec((C, C), lambda i: (0, 0)),
            pl.BlockSpec((C, 1), lambda i: (0, 0)),
            pl.BlockSpec((C, rb), lambda i: (0, i)),
        ],
        out_specs=pl.BlockSpec((C, rb), lambda i: (0, i)),
        out_shape=jax.ShapeDtypeStruct((C, NPAD * T), jnp.float32),
    )(z_r, weight, bias, x2p)


def kernel(x, weight, bias, filter_rows, filter_cols, filter_vals):
    xrows = x.reshape(N * T, C)
    rows = jnp.concatenate(
        [filter_rows.astype(jnp.int32),
         jnp.full((NNZ_PAD - NNZ,), NPAD - 1, jnp.int32)])
    cols = jnp.concatenate(
        [filter_cols.astype(jnp.int32), jnp.zeros((NNZ_PAD - NNZ,), jnp.int32)])
    vals = jnp.concatenate(
        [filter_vals, jnp.zeros((NNZ_PAD - NNZ,), jnp.float32)])

    z_t = _sc_spmm(xrows, rows, cols, vals)
    z_r = z_t.transpose(1, 0, 2).reshape(NPAD * T, C)

    x2p = jnp.pad(x.reshape(C, N * T), ((0, 0), (0, (NPAD - N) * T)))
    out2 = _tc_post(z_r, weight, bias.reshape(C, 1), x2p)
    return out2[:, :N * T].reshape(1, C, T, N)


# speedup vs baseline: 2.1611x; 1.0734x over previous
import functools

import jax
import jax.numpy as jnp
from jax import lax
from jax.experimental import pallas as pl
from jax.experimental.pallas import tpu as pltpu
from jax.experimental.pallas import tpu_sc as plsc

N = 10000
NPAD = 10240
T = 12
C = 128
NNZ = 160000
NCORES = 2
NSUB = 16
EBLK = 128
NBLK = 80
EPT = EBLK * NBLK
NNZ_PAD = EPT * NSUB
CHUNKS_PER_CORE = T // NCORES
ROWS_PER_TILE = NPAD // NSUB


def _sc_spmm_body(xrows, rows, cols, vals, z,
                  gbuf0, gbuf1, colsb0, colsb1, valsb0, valsb1,
                  rowsb0, rowsb1, rowscat0, rowscat1, acc,
                  ge0, ge1, se0, se1, lc0, lc1, lv0, lv1, lr0, lr1):
    c = lax.axis_index("c")
    s = lax.axis_index("s")
    e0 = s * EPT
    gbufs = [gbuf0, gbuf1]
    colsbx = [colsb0, colsb1]
    valsbx = [valsb0, valsb1]
    rowsbx = [rowsb0, rowsb1]
    rowscats = [rowscat0, rowscat1]
    ges = [ge0, ge1]
    ses = [se0, se1]
    lcs = [lc0, lc1]
    lvs = [lv0, lv1]
    lrs = [lr0, lr1]

    def fire_load(bv, p):
        off = e0 + bv * EBLK
        pltpu.async_copy(cols.at[pl.ds(off, EBLK)], colsbx[p], lcs[p])
        pltpu.async_copy(vals.at[pl.ds(off, EBLK)], valsbx[p], lvs[p])
        pltpu.async_copy(rows.at[pl.ds(off, EBLK)], rowsbx[p], lrs[p])

    def wait_load(p):
        pltpu.make_async_copy(cols.at[pl.ds(0, EBLK)], colsbx[p], lcs[p]).wait()
        pltpu.make_async_copy(vals.at[pl.ds(0, EBLK)], valsbx[p], lvs[p]).wait()
        pltpu.make_async_copy(rows.at[pl.ds(0, EBLK)], rowsbx[p], lrs[p]).wait()

    def fire_gather(p):
        pltpu.async_copy(xrows.at[colsbx[p]], gbufs[p], ges[p])

    def wait_gather(p):
        pltpu.make_async_copy(xrows.at[colsbx[p]], gbufs[p], ges[p]).wait()

    def fire_scatter(p):
        pltpu.async_copy(gbufs[p], acc.at[rowscats[p]], ses[p], add=True)

    def wait_scatter(p):
        pltpu.make_async_copy(gbufs[p], acc.at[rowscats[p]], ses[p]).wait()

    def chunk_body(jj, _):
        j = c * CHUNKS_PER_CORE + jj

        def idx_compute(p):
            def f(g, _):
                colsbx[p][pl.ds(g * 16, 16)] = colsbx[p][pl.ds(g * 16, 16)] * T + j
                return 0
            lax.fori_loop(0, EBLK // 16, f, 0)

        def scale(p):
            def g_body(g, _):
                rowscats[p][pl.ds(g * 16, 16)] = rowsbx[p][pl.ds(g * 16, 16)]
                v16 = valsbx[p][pl.ds(g * 16, 16)]
                for i in range(16):
                    vb = jnp.broadcast_to(v16[i:i + 1], (16,))
                    e = g * 16 + i
                    for k in range(C // 16):
                        gbufs[p][e, pl.ds(k * 16, 16)] = (
                            gbufs[p][e, pl.ds(k * 16, 16)] * vb)
                return 0
            lax.fori_loop(0, EBLK // 16, g_body, 0)

        def slot(bv, p, wait_prev_scatter, next_gather, next_load):
            q = 1 - p
            if next_gather:
                wait_load(q)
                idx_compute(q)
                if wait_prev_scatter:
                    wait_scatter(q)
                fire_gather(q)
            wait_gather(p)
            scale(p)
            fire_scatter(p)
            if next_load:
                fire_load(bv + 2, p)

        def _zfill(i, _):
            for k in range(C // 16):
                gbuf0[i, pl.ds(k * 16, 16)] = jnp.zeros((16,), jnp.float32)
            return 0
        lax.fori_loop(0, EBLK, _zfill, 0)
        for p in range(ROWS_PER_TILE // EBLK):
            pltpu.sync_copy(gbuf0, acc.at[pl.ds(s * ROWS_PER_TILE + p * EBLK, EBLK)])
        plsc.subcore_barrier()

        fire_load(0, 0)
        fire_load(1, 1)
        wait_load(0)
        idx_compute(0)
        fire_gather(0)
        slot(0, 0, wait_prev_scatter=False, next_gather=True, next_load=True)
        slot(1, 1, wait_prev_scatter=True, next_gather=True, next_load=True)

        def pair(ii, _):
            bv = 2 * ii + 2
            slot(bv, 0, wait_prev_scatter=True, next_gather=True, next_load=True)
            slot(bv + 1, 1, wait_prev_scatter=True, next_gather=True, next_load=True)
            return 0
        lax.fori_loop(0, (NBLK - 4) // 2, pair, 0)

        slot(NBLK - 2, 0, wait_prev_scatter=True, next_gather=True, next_load=False)
        slot(NBLK - 1, 1, wait_prev_scatter=False, next_gather=False, next_load=False)
        wait_scatter(0)
        wait_scatter(1)

        plsc.subcore_barrier()
        pltpu.sync_copy(acc.at[pl.ds(s * ROWS_PER_TILE, ROWS_PER_TILE)],
                        z.at[pl.ds(s * ROWS_PER_TILE, ROWS_PER_TILE), j])
        plsc.subcore_barrier()
        return 0

    lax.fori_loop(0, CHUNKS_PER_CORE, chunk_body, 0)


def _sc_spmm(xrows, rows, cols, vals):
    mesh = plsc.VectorSubcoreMesh(core_axis_name="c", subcore_axis_name="s",
                                  num_cores=NCORES, num_subcores=NSUB)
    f = pl.kernel(
        _sc_spmm_body,
        out_type=jax.ShapeDtypeStruct((NPAD, T, C), jnp.float32),
        mesh=mesh,
        scratch_types=[
            pltpu.VMEM((EBLK, C), jnp.float32),
            pltpu.VMEM((EBLK, C), jnp.float32),
            pltpu.VMEM((EBLK,), jnp.int32),
            pltpu.VMEM((EBLK,), jnp.int32),
            pltpu.VMEM((EBLK,), jnp.float32),
            pltpu.VMEM((EBLK,), jnp.float32),
            pltpu.VMEM((EBLK,), jnp.int32),
            pltpu.VMEM((EBLK,), jnp.int32),
            pltpu.VMEM((EBLK,), jnp.int32),
            pltpu.VMEM((EBLK,), jnp.int32),
            pltpu.VMEM_SHARED((NPAD, C), jnp.float32),
        ] + [pltpu.SemaphoreType.DMA] * 10,
    )
    return f(xrows, rows, cols, vals)


def _tc_body(z_ref, w_ref, b_ref, x_ref, o_ref):
    g = lax.dot_general(w_ref[...], z_ref[...], (((0,), (1,)), ((), ())),
                        preferred_element_type=jnp.float32)
    o_ref[...] = g + b_ref[...] + x_ref[...]


def _tc_post(z_r, weight, bias, x2):
    rb = T * C
    grid = pl.cdiv(N * T, rb)
    return pl.pallas_call(
        _tc_body,
        grid=(grid,),
        in_specs=[
            pl.BlockSpec((rb, C), lambda i: (i, 0)),
            pl.BlockSpec((C, C), lambda i: (0, 0)),
            pl.BlockSpec((C, 1), lambda i: (0, 0)),
            pl.BlockSpec((C, rb), lambda i: (0, i)),
        ],
        out_specs=pl.BlockSpec((C, rb), lambda i: (0, i)),
        out_shape=jax.ShapeDtypeStruct((C, N * T), jnp.float32),
    )(z_r, weight, bias, x2)


def kernel(x, weight, bias, filter_rows, filter_cols, filter_vals):
    xrows = x.reshape(N * T, C)
    rows = jnp.concatenate(
        [filter_rows.astype(jnp.int32),
         jnp.full((NNZ_PAD - NNZ,), NPAD - 1, jnp.int32)])
    cols = jnp.concatenate(
        [filter_cols.astype(jnp.int32), jnp.zeros((NNZ_PAD - NNZ,), jnp.int32)])
    vals = jnp.concatenate(
        [filter_vals, jnp.zeros((NNZ_PAD - NNZ,), jnp.float32)])

    z_t = _sc_spmm(xrows, rows, cols, vals)
    z_r = z_t.reshape(NPAD * T, C)

    x2 = x.reshape(C, N * T)
    out2 = _tc_post(z_r, weight, bias.reshape(C, 1), x2)
    return out2.reshape(1, C, T, N)
